# Initial kernel scaffold; baseline (speedup 1.0000x reference)
import jax
import jax.numpy as jnp
from jax import lax
from jax.experimental import pallas as pl
from jax.experimental.pallas import tpu as pltpu
from jax.experimental.pallas import tpu_sc as plsc

_P = 8
_C = 32
_CHUNK = 128
_NW = 32
_L = 16


def _sc_body(edges, table, wrow_in, ratios, wout,
             idx_i, idx_j, rows_i, rows_j, out_v, wrow,
             sem_a, sem_b, sem_w):
    total_chunks = edges.shape[1] // _CHUNK
    wid = lax.axis_index("s") * 2 + lax.axis_index("c")
    pltpu.sync_copy(wrow_in, wrow)
    iota = lax.broadcasted_iota(jnp.int32, (_L,), 0)
    nchunks = (total_chunks - wid + _NW - 1) // _NW

    def chunk_body(k, carry):
        c = wid + k * _NW
        base = c * _CHUNK
        cp_i = pltpu.async_copy(edges.at[0, pl.ds(base, _CHUNK)], idx_i, sem_a)
        cp_j = pltpu.async_copy(edges.at[1, pl.ds(base, _CHUNK)], idx_j, sem_b)
        cp_w = pltpu.async_copy(wrow, wout.at[pl.ds(base * _P, _CHUNK * _P)],
                                sem_w)
        cp_i.wait()
        cp_j.wait()
        g_i = pltpu.async_copy(table.at[idx_i], rows_i, sem_a)
        g_j = pltpu.async_copy(table.at[idx_j], rows_j, sem_b)
        g_i.wait()
        g_j.wait()

        def g_body(g, carry2):
            e0 = g * _L + iota
            for p in range(_P):
                c0 = jnp.full((_L,), 4 * p, jnp.int32)
                c1 = c0 + 1
                c2 = c0 + 2
                c3 = c0 + 3
                pw = plsc.load_gather(rows_i, [e0, c0])
                px = plsc.load_gather(rows_i, [e0, c1])
                py = plsc.load_gather(rows_i, [e0, c2])
                pz = plsc.load_gather(rows_i, [e0, c3])
                qw = plsc.load_gather(rows_j, [e0, c0])
                qx = plsc.load_gather(rows_j, [e0, c1])
                qy = plsc.load_gather(rows_j, [e0, c2])
                qz = plsc.load_gather(rows_j, [e0, c3])
                r = 1.0 / (qw * qw + qx * qx + qy * qy + qz * qz)
                ow = (pw * qw + px * qx + py * qy + pz * qz) * r
                ox = (px * qw - pw * qx - py * qz + pz * qy) * r
                oy = (py * qw - pw * qy + px * qz - pz * qx) * r
                oz = (pz * qw - pw * qz - px * qy + py * qx) * r
                plsc.store_scatter(out_v, [e0, c0], ow)
                plsc.store_scatter(out_v, [e0, c1], ox)
                plsc.store_scatter(out_v, [e0, c2], oy)
                plsc.store_scatter(out_v, [e0, c3], oz)
            return carry2

        lax.fori_loop(0, _CHUNK // _L, g_body, 0)
        pltpu.sync_copy(out_v, ratios.at[pl.ds(base, _CHUNK), :])
        cp_w.wait()
        return carry

    lax.fori_loop(0, nchunks, chunk_body, 0)


def kernel(particles, weights, edges):
    n_nodes = particles.shape[0]
    e = edges.shape[1]
    table = particles.reshape(n_nodes, _C)
    wrow_in = jnp.tile(weights[0], _CHUNK)
    call = pl.kernel(
        _sc_body,
        out_type=[
            jax.ShapeDtypeStruct((e, _C), jnp.float32),
            jax.ShapeDtypeStruct((e * _P,), jnp.float32),
        ],
        mesh=plsc.VectorSubcoreMesh(core_axis_name="c", subcore_axis_name="s"),
        scratch_types=[
            pltpu.VMEM((_CHUNK,), jnp.int32),
            pltpu.VMEM((_CHUNK,), jnp.int32),
            pltpu.VMEM((_CHUNK, _C), jnp.float32),
            pltpu.VMEM((_CHUNK, _C), jnp.float32),
            pltpu.VMEM((_CHUNK, _C), jnp.float32),
            pltpu.VMEM((_CHUNK * _P,), jnp.float32),
            pltpu.SemaphoreType.DMA,
            pltpu.SemaphoreType.DMA,
            pltpu.SemaphoreType.DMA,
        ],
    )
    ratios2d, wflat = call(edges, table, wrow_in)
    return ratios2d.reshape(e, _P, 4), wflat.reshape(e, _P)


# speedup vs baseline: 7.3842x; 7.3842x over previous
import jax
import jax.numpy as jnp
from jax import lax
from jax.experimental import pallas as pl
from jax.experimental.pallas import tpu as pltpu
from jax.experimental.pallas import tpu_sc as plsc

_P = 8
_C = 32
_CHUNK = 128
_NW = 32
_L = 16


def _sc_body(edges, table, wrow_in, ratios, wout,
             idx_i, idx_j, rows_i, rows_j, out_v, wrow,
             sem_a, sem_b, sem_w):
    total_chunks = edges.shape[1] // _CHUNK
    wid = lax.axis_index("s") * 2 + lax.axis_index("c")
    pltpu.sync_copy(wrow_in, wrow)
    iota = lax.broadcasted_iota(jnp.int32, (_L,), 0)
    nchunks = (total_chunks - wid + _NW - 1) // _NW

    def chunk_body(k, carry):
        c = wid + k * _NW
        base = c * _CHUNK
        cp_i = pltpu.async_copy(edges.at[0, pl.ds(base, _CHUNK)], idx_i, sem_a)
        cp_j = pltpu.async_copy(edges.at[1, pl.ds(base, _CHUNK)], idx_j, sem_b)
        cp_w = pltpu.async_copy(wrow, wout.at[pl.ds(base * _P, _CHUNK * _P)],
                                sem_w)
        cp_i.wait()
        cp_j.wait()
        g_i = pltpu.async_copy(table.at[idx_i], rows_i, sem_a)
        g_j = pltpu.async_copy(table.at[idx_j], rows_j, sem_b)
        g_i.wait()
        g_j.wait()

        def g_body(g, carry2):
            e0 = g * _L + iota
            for p in range(_P):
                c0 = jnp.full((_L,), 4 * p, jnp.int32)
                c1 = c0 + 1
                c2 = c0 + 2
                c3 = c0 + 3
                pw = plsc.load_gather(rows_i, [e0, c0])
                px = plsc.load_gather(rows_i, [e0, c1])
                py = plsc.load_gather(rows_i, [e0, c2])
                pz = plsc.load_gather(rows_i, [e0, c3])
                qw = plsc.load_gather(rows_j, [e0, c0])
                qx = plsc.load_gather(rows_j, [e0, c1])
                qy = plsc.load_gather(rows_j, [e0, c2])
                qz = plsc.load_gather(rows_j, [e0, c3])
                r = 1.0 / (qw * qw + qx * qx + qy * qy + qz * qz)
                ow = (pw * qw + px * qx + py * qy + pz * qz) * r
                ox = (px * qw - pw * qx - py * qz + pz * qy) * r
                oy = (py * qw - pw * qy + px * qz - pz * qx) * r
                oz = (pz * qw - pw * qz - px * qy + py * qx) * r
                plsc.store_scatter(out_v, [e0, c0], ow)
                plsc.store_scatter(out_v, [e0, c1], ox)
                plsc.store_scatter(out_v, [e0, c2], oy)
                plsc.store_scatter(out_v, [e0, c3], oz)
            return carry2

        lax.fori_loop(0, _CHUNK // _L, g_body, 0)
        pltpu.sync_copy(out_v, ratios.at[pl.ds(base, _CHUNK), :])
        cp_w.wait()
        return carry

    lax.fori_loop(0, nchunks, chunk_body, 0)


def kernel(particles, weights, edges):
    n_nodes = particles.shape[0]
    e = edges.shape[1]
    table = particles.reshape(n_nodes, _C)
    wrow_in = jnp.tile(weights[0], _CHUNK)
    call = pl.kernel(
        _sc_body,
        out_type=[
            jax.ShapeDtypeStruct((e, _C), jnp.float32),
            jax.ShapeDtypeStruct((e * _P,), jnp.float32),
        ],
        mesh=plsc.VectorSubcoreMesh(core_axis_name="c", subcore_axis_name="s"),
        compiler_params=pltpu.CompilerParams(
            needs_layout_passes=False, use_tc_tiling_on_sc=False
        ),
        scratch_types=[
            pltpu.VMEM((_CHUNK,), jnp.int32),
            pltpu.VMEM((_CHUNK,), jnp.int32),
            pltpu.VMEM((_CHUNK, _C), jnp.float32),
            pltpu.VMEM((_CHUNK, _C), jnp.float32),
            pltpu.VMEM((_CHUNK, _C), jnp.float32),
            pltpu.VMEM((_CHUNK * _P,), jnp.float32),
            pltpu.SemaphoreType.DMA,
            pltpu.SemaphoreType.DMA,
            pltpu.SemaphoreType.DMA,
        ],
    )
    ratios2d, wflat = call(edges, table, wrow_in)
    return ratios2d.reshape(e, _P, 4), wflat.reshape(e, _P)


# speedup vs baseline: 8.5584x; 1.1590x over previous
import jax
import jax.numpy as jnp
from jax import lax
from jax.experimental import pallas as pl
from jax.experimental.pallas import tpu as pltpu
from jax.experimental.pallas import tpu_sc as plsc

_P = 8
_C = 32
_G = 128
_SUB = 4
_CHUNK = _G * _SUB
_NW = 32
_L = 16


def _sc_body(edges, table, wrow_in, ratios, wout,
             idx_i0, idx_i1, idx_j0, idx_j1,
             rows_i0, rows_i1, rows_j0, rows_j1,
             out0, out1, wrow,
             sem_idx0, sem_idx1, sem_g0, sem_g1,
             sem_o0, sem_o1, sem_w0, sem_w1):
    total = edges.shape[1]
    idx_i = [idx_i0, idx_i1]
    idx_j = [idx_j0, idx_j1]
    rows_i = [rows_i0, rows_i1]
    rows_j = [rows_j0, rows_j1]
    out = [out0, out1]
    sem_idx = [sem_idx0, sem_idx1]
    sem_g = [sem_g0, sem_g1]
    sem_o = [sem_o0, sem_o1]
    sem_w = [sem_w0, sem_w1]

    wid = lax.axis_index("s") * 2 + lax.axis_index("c")
    pltpu.sync_copy(wrow_in, wrow)
    iota = lax.broadcasted_iota(jnp.int32, (_L,), 0)

    kmax = (total + _NW - 1) // _NW
    kmax += kmax % 2
    nsteps = kmax // 2

    def fire_idx(b, c):
        pltpu.async_copy(edges.at[0, c], idx_i[b], sem_idx[b])
        pltpu.async_copy(edges.at[1, c], idx_j[b], sem_idx[b])

    def wait_idx(b, c):
        pltpu.make_async_copy(edges.at[0, c], idx_i[b], sem_idx[b]).wait()
        pltpu.make_async_copy(edges.at[1, c], idx_j[b], sem_idx[b]).wait()

    def fire_gathers(b):
        for r in range(_SUB):
            dst_i = rows_i[b].at[pl.ds(r * _G, _G), :]
            dst_j = rows_j[b].at[pl.ds(r * _G, _G), :]
            pltpu.async_copy(table.at[idx_i[b].at[r]], dst_i, sem_g[b])
            pltpu.async_copy(table.at[idx_j[b].at[r]], dst_j, sem_g[b])

    def wait_gathers(b):
        for r in range(_SUB):
            dst_i = rows_i[b].at[pl.ds(r * _G, _G), :]
            dst_j = rows_j[b].at[pl.ds(r * _G, _G), :]
            pltpu.make_async_copy(table.at[idx_i[b].at[r]], dst_i,
                                  sem_g[b]).wait()
            pltpu.make_async_copy(table.at[idx_j[b].at[r]], dst_j,
                                  sem_g[b]).wait()

    def fire_out(b, c):
        pltpu.async_copy(out[b], ratios.at[pl.ds(c * _CHUNK, _CHUNK), :],
                         sem_o[b])
        pltpu.async_copy(wrow, wout.at[pl.ds(c * _CHUNK * _P, _CHUNK * _P)],
                         sem_w[b])

    def wait_out(b, c):
        pltpu.make_async_copy(out[b], ratios.at[pl.ds(c * _CHUNK, _CHUNK), :],
                              sem_o[b]).wait()
        pltpu.make_async_copy(wrow,
                              wout.at[pl.ds(c * _CHUNK * _P, _CHUNK * _P)],
                              sem_w[b]).wait()

    def compute(b):
        def g_body(g, carry):
            e0 = g * _L + iota
            for p in range(_P):
                c0 = jnp.full((_L,), 4 * p, jnp.int32)
                c1 = c0 + 1
                c2 = c0 + 2
                c3 = c0 + 3
                pw = plsc.load_gather(rows_i[b], [e0, c0])
                px = plsc.load_gather(rows_i[b], [e0, c1])
                py = plsc.load_gather(rows_i[b], [e0, c2])
                pz = plsc.load_gather(rows_i[b], [e0, c3])
                qw = plsc.load_gather(rows_j[b], [e0, c0])
                qx = plsc.load_gather(rows_j[b], [e0, c1])
                qy = plsc.load_gather(rows_j[b], [e0, c2])
                qz = plsc.load_gather(rows_j[b], [e0, c3])
                r = 1.0 / (qw * qw + qx * qx + qy * qy + qz * qz)
                ow = (pw * qw + px * qx + py * qy + pz * qz) * r
                ox = (px * qw - pw * qx - py * qz + pz * qy) * r
                oy = (py * qw - pw * qy + px * qz - pz * qx) * r
                oz = (pz * qw - pw * qz - px * qy + py * qx) * r
                plsc.store_scatter(out[b], [e0, c0], ow)
                plsc.store_scatter(out[b], [e0, c1], ox)
                plsc.store_scatter(out[b], [e0, c2], oy)
                plsc.store_scatter(out[b], [e0, c3], oz)
            return carry

        lax.fori_loop(0, _CHUNK // _L, g_body, 0)

    for b in range(2):
        fire_idx(b, wid + b * _NW)
    for b in range(2):
        wait_idx(b, wid + b * _NW)
        fire_gathers(b)

    def step(t, carry):
        for b in range(2):
            k = 2 * t + b
            c = wid + k * _NW
            cn = c + 2 * _NW

            def process():
                wait_gathers(b)

                @pl.when(cn < total)
                def _():
                    fire_idx(b, cn)

                @pl.when(t >= 1)
                def _():
                    wait_out(b, c - 2 * _NW)

                compute(b)
                fire_out(b, c)

                @pl.when(cn < total)
                def _():
                    wait_idx(b, cn)
                    fire_gathers(b)

            if b == 0:
                process()
            else:
                pl.when(c < total)(process)
        return carry

    lax.fori_loop(0, nsteps, step, 0)

    c_last0 = wid + (kmax - 2) * _NW
    c_last1 = wid + (kmax - 1) * _NW
    wait_out(0, c_last0)

    @pl.when(c_last1 < total)
    def _():
        wait_out(1, c_last1)


def kernel(particles, weights, edges):
    n_nodes = particles.shape[0]
    e = edges.shape[1]
    n_macro = e // _CHUNK
    table = particles.reshape(n_nodes, _C)
    edges4 = edges.reshape(2, n_macro, _SUB, _G)
    wrow_in = jnp.tile(weights[0], _CHUNK)
    call = pl.kernel(
        _sc_body,
        out_type=[
            jax.ShapeDtypeStruct((e, _C), jnp.float32),
            jax.ShapeDtypeStruct((e * _P,), jnp.float32),
        ],
        mesh=plsc.VectorSubcoreMesh(core_axis_name="c", subcore_axis_name="s"),
        compiler_params=pltpu.CompilerParams(
            needs_layout_passes=False, use_tc_tiling_on_sc=False
        ),
        scratch_types=[
            pltpu.VMEM((_SUB, _G), jnp.int32),
            pltpu.VMEM((_SUB, _G), jnp.int32),
            pltpu.VMEM((_SUB, _G), jnp.int32),
            pltpu.VMEM((_SUB, _G), jnp.int32),
            pltpu.VMEM((_CHUNK, _C), jnp.float32),
            pltpu.VMEM((_CHUNK, _C), jnp.float32),
            pltpu.VMEM((_CHUNK, _C), jnp.float32),
            pltpu.VMEM((_CHUNK, _C), jnp.float32),
            pltpu.VMEM((_CHUNK, _C), jnp.float32),
            pltpu.VMEM((_CHUNK, _C), jnp.float32),
            pltpu.VMEM((_CHUNK * _P,), jnp.float32),
            pltpu.SemaphoreType.DMA,
            pltpu.SemaphoreType.DMA,
            pltpu.SemaphoreType.DMA,
            pltpu.SemaphoreType.DMA,
            pltpu.SemaphoreType.DMA,
            pltpu.SemaphoreType.DMA,
            pltpu.SemaphoreType.DMA,
            pltpu.SemaphoreType.DMA,
        ],
    )
    ratios2d, wflat = call(edges4, table, wrow_in)
    return ratios2d.reshape(e, _P, 4), wflat.reshape(e, _P)
